# Initial kernel scaffold; baseline (speedup 1.0000x reference)
import jax
import jax.numpy as jnp
from jax.experimental import pallas as pl


def kernel(x, h, y, perm_xy, perm_xh, perm_hy):
    raise NotImplementedError("write your pallas kernel here")


# speedup vs baseline: 28.7946x; 28.7946x over previous
import functools
import math

import jax
import jax.numpy as jnp
from jax.experimental import pallas as pl
from jax.experimental.pallas import tpu as pltpu

_ALPHA = 0.001
_SPARSITY = 0.05


def _normalize_kernel(pre_ref, post_ref, perm_ref, out_ref):
    prod = jax.lax.dot_general(
        pre_ref[...], post_ref[...], (((0,), (0,)), ((), ())),
        preferred_element_type=jnp.float32)
    p = perm_ref[...] + _ALPHA * prod
    s = jnp.sum(p, axis=0, keepdims=True)
    out_ref[...] = p / s


def _select_stats_kernel(k, n_chunks, pn_ref, t_ref, c_ref):
    n_rows, n_cols = pn_ref.shape
    r = n_rows // n_chunks

    def chunk_u(ci):
        sl = pl.ds(pl.multiple_of(ci * r, r), r)
        return jax.lax.bitcast_convert_type(pn_ref[sl, :], jnp.int32)

    def count_ge(trial):
        def body(ci, acc):
            return acc + jnp.sum((chunk_u(ci) >= trial).astype(jnp.int32))
        return jax.lax.fori_loop(0, n_chunks, body, jnp.int32(0))

    def bit_body(i, t):
        trial = jnp.bitwise_or(t, jax.lax.shift_left(jnp.int32(1), jnp.int32(30) - i))
        return jnp.where(count_ge(trial) >= k, trial, t)

    t = jax.lax.fori_loop(0, 31, bit_body, jnp.int32(0))

    c1 = count_ge(t + 1)
    c2 = count_ge(t) - c1
    need = k - c1

    def count_eq_lt(cut):
        def body(ci, acc):
            u = chunk_u(ci)
            rows = jax.lax.broadcasted_iota(jnp.int32, (r, n_cols), 0)
            cols = jax.lax.broadcasted_iota(jnp.int32, (r, n_cols), 1)
            flat = (rows + ci * r) * n_cols + cols
            return acc + jnp.sum(((u == t) & (flat < cut)).astype(jnp.int32))
        return jax.lax.fori_loop(0, n_chunks, body, jnp.int32(0))

    def exact():
        def c_body(i, c):
            trial = jnp.bitwise_or(c, jax.lax.shift_left(jnp.int32(1), jnp.int32(23) - i))
            return jnp.where(count_eq_lt(trial) <= need, trial, c)
        return jax.lax.fori_loop(0, 24, c_body, jnp.int32(0))

    c = jax.lax.cond(c2 == need, lambda: jnp.int32(1 << 24), exact)
    t_ref[0, 0] = t
    c_ref[0, 0] = c


def _mask_kernel(row_block, t_ref, c_ref, pn_ref, mask_ref):
    u = jax.lax.bitcast_convert_type(pn_ref[...], jnp.int32)
    t = t_ref[0, 0]
    cut = c_ref[0, 0]
    n_cols = u.shape[1]
    rows = jax.lax.broadcasted_iota(jnp.int32, u.shape, 0)
    cols = jax.lax.broadcasted_iota(jnp.int32, u.shape, 1)
    flat = (rows + pl.program_id(0) * row_block) * n_cols + cols
    mask_ref[...] = ((u > t) | ((u == t) & (flat < cut))).astype(jnp.int32)


def _update_one(pre, post, perm, col_block=512, row_block=256, n_chunks=16):
    n_pre, n_post = perm.shape
    b = pre.shape[0]
    k = math.ceil(n_pre * n_post * _SPARSITY)
    nc = n_post // col_block
    nr = n_pre // row_block

    pn = pl.pallas_call(
        _normalize_kernel,
        grid=(nc,),
        in_specs=[
            pl.BlockSpec((b, n_pre), lambda j: (0, 0)),
            pl.BlockSpec((b, col_block), lambda j: (0, j)),
            pl.BlockSpec((n_pre, col_block), lambda j: (0, j)),
        ],
        out_specs=pl.BlockSpec((n_pre, col_block), lambda j: (0, j)),
        out_shape=jax.ShapeDtypeStruct((n_pre, n_post), jnp.float32),
    )(pre, post, perm)

    t, c = pl.pallas_call(
        functools.partial(_select_stats_kernel, k, n_chunks),
        out_specs=[
            pl.BlockSpec(memory_space=pltpu.SMEM),
            pl.BlockSpec(memory_space=pltpu.SMEM),
        ],
        out_shape=[
            jax.ShapeDtypeStruct((1, 1), jnp.int32),
            jax.ShapeDtypeStruct((1, 1), jnp.int32),
        ],
    )(pn)

    mask = pl.pallas_call(
        functools.partial(_mask_kernel, row_block),
        grid=(nr,),
        in_specs=[
            pl.BlockSpec(memory_space=pltpu.SMEM),
            pl.BlockSpec(memory_space=pltpu.SMEM),
            pl.BlockSpec((row_block, n_post), lambda i: (i, 0)),
        ],
        out_specs=pl.BlockSpec((row_block, n_post), lambda i: (i, 0)),
        out_shape=jax.ShapeDtypeStruct((n_pre, n_post), jnp.int32),
    )(t, c, pn)
    return pn, mask


def kernel(x, h, y, perm_xy, perm_xh, perm_hy):
    pn_xy, w_xy = _update_one(x, y, perm_xy)
    pn_xh, w_xh = _update_one(x, h, perm_xh)
    pn_hy, w_hy = _update_one(h, y, perm_hy)
    return (w_xy, w_xh, w_hy, pn_xy, pn_xh, pn_hy)
